# baseline (speedup 1.0000x reference)
import functools

import numpy as np

import jax
import jax.numpy as jnp
from jax import lax
from jax.experimental import pallas as pl
from jax.experimental.pallas import tpu as pltpu
from jax.experimental.pallas import tpu_sc as plsc

_VOCAB = 1_000_000
_D = 64
_BATCH = 4096
_SEQ = 200

_TBLK = 128
_NW = 32
_RPW = _BATCH // _NW
_IPW = _SEQ * _RPW

_SCSTART = 507_904
_SCEND = 999_424
_SCMID = _SCEND - _SCSTART
_SCROWS = _SCMID // _NW
_SCWIN = 256
_NWIN = _SCROWS // _SCWIN

_HTLEN = _SCSTART + 16384
_ZHT = _HTLEN
_PROWS = _SCMID // 8
_ZMID = 8


def _proj_body(tbl_ref, w_ref, out_ref):
    blk = tbl_ref[...]
    w = w_ref[0, :]
    r3 = blk.reshape(_TBLK, 128, _D)
    out_ref[...] = jnp.sum(r3 * w[None, None, :], axis=2)


def _project_tc(table, w, nblk, blk0):
    return pl.pallas_call(
        _proj_body,
        grid=(nblk,),
        in_specs=[
            pl.BlockSpec((_TBLK * 128, _D), lambda i: (blk0 + i, 0)),
            pl.BlockSpec((1, _D), lambda i: (0, 0)),
        ],
        out_specs=pl.BlockSpec((_TBLK, 128), lambda i: (i, 0)),
        out_shape=jax.ShapeDtypeStruct((nblk * _TBLK, 128), jnp.float32),
    )(table, w)


def _scproj_body(tbl_hbm, wv_hbm, out_hbm, buf0, buf1, p_v, w_v, sem0, sem1):
    wid = lax.axis_index("s") * 2 + lax.axis_index("c")
    rbase = _SCSTART + wid * _SCROWS
    pltpu.sync_copy(wv_hbm, w_v)
    wk = [w_v[pl.ds(k * 16, 16)] for k in range(4)]

    def fire(c, buf, sem):
        pltpu.make_async_copy(
            tbl_hbm.at[pl.ds(rbase + c * _SCWIN, _SCWIN), :], buf, sem).start()

    def wait(c, buf, sem):
        pltpu.make_async_copy(
            tbl_hbm.at[pl.ds(rbase + c * _SCWIN, _SCWIN), :], buf, sem).wait()

    def compute(c, buf):
        def grp_body(g, inner):
            for r in range(16):
                row = g * 16 + r
                p = (buf[row, pl.ds(0, 16)] * wk[0]
                     + buf[row, pl.ds(16, 16)] * wk[1]
                     + buf[row, pl.ds(32, 16)] * wk[2]
                     + buf[row, pl.ds(48, 16)] * wk[3])
                p_v[pl.ds(g * 256 + r * 16, 16)] = p
            return inner

        lax.fori_loop(0, _SCWIN // 16, grp_body, 0)
        pltpu.sync_copy(
            p_v,
            out_hbm.at[pl.ds((wid * _SCROWS + c * _SCWIN) * 16, _SCWIN * 16)])

    fire(0, buf0, sem0)

    def main(i, carry):
        c0 = 2 * i
        c1 = 2 * i + 1
        fire(c1, buf1, sem1)
        wait(c0, buf0, sem0)
        compute(c0, buf0)

        @pl.when(i < _NWIN // 2 - 1)
        def _():
            fire(c0 + 2, buf0, sem0)

        wait(c1, buf1, sem1)
        compute(c1, buf1)
        return carry

    lax.fori_loop(0, _NWIN // 2, main, 0)


def _project_sc(table, wb):
    mesh = plsc.VectorSubcoreMesh(core_axis_name="c", subcore_axis_name="s")
    fn = pl.kernel(
        _scproj_body,
        mesh=mesh,
        out_type=jax.ShapeDtypeStruct((_SCMID * 16,), jnp.float32),
        scratch_types=[
            pltpu.VMEM((_SCWIN, _D), jnp.float32),
            pltpu.VMEM((_SCWIN, _D), jnp.float32),
            pltpu.VMEM((_SCWIN * 16,), jnp.float32),
            pltpu.VMEM((_D,), jnp.float32),
            pltpu.SemaphoreType.DMA,
            pltpu.SemaphoreType.DMA,
        ],
    )
    return fn(table, wb)


def _fold_body(p_ref, m_ref, out_ref):
    out_ref[...] = jax.lax.dot_general(
        p_ref[...], m_ref[...], (((1,), (0,)), ((), ())),
        preferred_element_type=jnp.float32)


def _fold16(partials2d, mconst):
    return pl.pallas_call(
        _fold_body,
        grid=(_PROWS // 2048,),
        in_specs=[
            pl.BlockSpec((2048, 128), lambda i: (i, 0)),
            pl.BlockSpec((128, 128), lambda i: (0, 0)),
        ],
        out_specs=pl.BlockSpec((2048, 128), lambda i: (i, 0)),
        out_shape=jax.ShapeDtypeStruct((_PROWS, 128), jnp.float32),
    )(partials2d, mconst)


_MFOLD = np.asarray(
    (np.arange(128)[:, None] // 16) == np.arange(128)[None, :],
    dtype=np.float32)


def _sc_body(xr1_hbm, xr2_hbm, ht_hbm, mid_hbm, bv_hbm, out_hbm,
             idx_v, g_v, res_v, b_v, sem):
    wid = lax.axis_index("s") * 2 + lax.axis_index("c")
    base = wid * _RPW
    pltpu.sync_copy(bv_hbm, b_v)
    zero = jnp.zeros((16,), jnp.float32)

    def body(s, accs):
        off = s * _RPW
        return tuple(
            accs[rb] + g_v[pl.ds(off + rb * 16, 16)] for rb in range(8)
        )

    pltpu.sync_copy(xr1_hbm.at[wid], idx_v)
    pltpu.async_copy(ht_hbm.at[idx_v], g_v, sem).wait()
    acc1 = lax.fori_loop(0, _SEQ, body, (zero,) * 8)
    pltpu.sync_copy(xr2_hbm.at[wid], idx_v)
    pltpu.async_copy(mid_hbm.at[idx_v], g_v, sem).wait()
    acc2 = lax.fori_loop(0, _SEQ, body, (zero,) * 8)

    inv = jnp.float32(1.0 / _SEQ)
    bb = b_v[...]
    for rb in range(8):
        z = (acc1[rb] + acc2[rb]) * inv + bb
        res_v[pl.ds(rb * 16, 16)] = 1.0 / (1.0 + jnp.exp(-z))
    pltpu.sync_copy(res_v, out_hbm.at[pl.ds(base, _RPW)])


def _gather_pool(xr1, xr2, ht, mid_flat, bvec):
    mesh = plsc.VectorSubcoreMesh(core_axis_name="c", subcore_axis_name="s")
    fn = pl.kernel(
        _sc_body,
        mesh=mesh,
        out_type=jax.ShapeDtypeStruct((_BATCH,), jnp.float32),
        scratch_types=[
            pltpu.VMEM((_IPW,), jnp.int32),
            pltpu.VMEM((_IPW,), jnp.float32),
            pltpu.VMEM((_RPW,), jnp.float32),
            pltpu.VMEM((16,), jnp.float32),
            pltpu.SemaphoreType.DMA,
        ],
    )
    return fn(xr1, xr2, ht, mid_flat, bvec)


def _rearrange(ix):
    return ix.reshape(_NW, _RPW, _SEQ).transpose(0, 2, 1).reshape(_NW, _IPW)


def kernel(x, table, W, b):
    wb = W.reshape(_D)
    partials = _project_sc(table, wb)
    t_head = _project_tc(table, W, 31, 0)
    t_tail = _project_tc(table, W, 1, 61)
    ht = jnp.concatenate(
        [t_head.reshape(-1), t_tail.reshape(-1), jnp.zeros((8,), jnp.float32)])
    mid = _fold16(partials.reshape(_PROWS, 128), jnp.asarray(_MFOLD))

    in_mid = (x >= _SCSTART) & (x < _SCEND)
    idx_ht = jnp.where(in_mid, _ZHT,
                       jnp.where(x < _SCSTART, x, x - _SCEND + _SCSTART))
    vp = x - _SCSTART
    idx_mid = jnp.where(in_mid, (vp >> 3) * 128 + (vp & 7), _ZMID)

    bvec = jnp.full((16,), b[0], jnp.float32)
    y = _gather_pool(_rearrange(idx_ht), _rearrange(idx_mid),
                     ht, mid.reshape(-1), bvec)
    return y.reshape(_BATCH, 1)


# speedup vs baseline: 6.7082x; 6.7082x over previous
import functools

import numpy as np

import jax
import jax.numpy as jnp
from jax import lax
from jax.experimental import pallas as pl
from jax.experimental.pallas import tpu as pltpu
from jax.experimental.pallas import tpu_sc as plsc

_VOCAB = 1_000_000
_D = 64
_BATCH = 4096
_SEQ = 200

_TBLK = 128
_NW = 32
_RPW = _BATCH // _NW
_IPW = _SEQ * _RPW

_SCSTART = 507_904
_SCEND = 999_424
_SCMID = _SCEND - _SCSTART
_SCROWS = _SCMID // _NW
_SCWIN = 256
_NWIN = _SCROWS // _SCWIN

_HTLEN = _SCSTART + 16384
_ZPAD = 8192
_PROWS = _SCMID // 8
_ZMID = 8


def _proj_body(tbl_ref, w_ref, out_ref):
    blk = tbl_ref[...]
    w = w_ref[0, :]
    r3 = blk.reshape(_TBLK, 128, _D)
    out_ref[...] = jnp.sum(r3 * w[None, None, :], axis=2)


def _project_tc(table, w, nblk, blk0):
    return pl.pallas_call(
        _proj_body,
        grid=(nblk,),
        in_specs=[
            pl.BlockSpec((_TBLK * 128, _D), lambda i: (blk0 + i, 0)),
            pl.BlockSpec((1, _D), lambda i: (0, 0)),
        ],
        out_specs=pl.BlockSpec((_TBLK, 128), lambda i: (i, 0)),
        out_shape=jax.ShapeDtypeStruct((nblk * _TBLK, 128), jnp.float32),
    )(table, w)


def _scproj_body(tbl_hbm, wv_hbm, out_hbm, buf0, buf1, p_v, w_v, sem0, sem1):
    wid = lax.axis_index("s") * 2 + lax.axis_index("c")
    rbase = _SCSTART + wid * _SCROWS
    pltpu.sync_copy(wv_hbm, w_v)
    wk = [w_v[pl.ds(k * 16, 16)] for k in range(4)]

    def fire(c, buf, sem):
        pltpu.make_async_copy(
            tbl_hbm.at[pl.ds(rbase + c * _SCWIN, _SCWIN), :], buf, sem).start()

    def wait(c, buf, sem):
        pltpu.make_async_copy(
            tbl_hbm.at[pl.ds(rbase + c * _SCWIN, _SCWIN), :], buf, sem).wait()

    def compute(c, buf):
        def grp_body(g, inner):
            for r in range(16):
                row = g * 16 + r
                p = (buf[row, pl.ds(0, 16)] * wk[0]
                     + buf[row, pl.ds(16, 16)] * wk[1]
                     + buf[row, pl.ds(32, 16)] * wk[2]
                     + buf[row, pl.ds(48, 16)] * wk[3])
                p_v[pl.ds(g * 256 + r * 16, 16)] = p
            return inner

        lax.fori_loop(0, _SCWIN // 16, grp_body, 0)
        pltpu.sync_copy(
            p_v,
            out_hbm.at[pl.ds((wid * _SCROWS + c * _SCWIN) * 16, _SCWIN * 16)])

    fire(0, buf0, sem0)

    def main(i, carry):
        c0 = 2 * i
        c1 = 2 * i + 1
        fire(c1, buf1, sem1)
        wait(c0, buf0, sem0)
        compute(c0, buf0)

        @pl.when(i < _NWIN // 2 - 1)
        def _():
            fire(c0 + 2, buf0, sem0)

        wait(c1, buf1, sem1)
        compute(c1, buf1)
        return carry

    lax.fori_loop(0, _NWIN // 2, main, 0)


def _project_sc(table, wb):
    mesh = plsc.VectorSubcoreMesh(core_axis_name="c", subcore_axis_name="s")
    fn = pl.kernel(
        _scproj_body,
        mesh=mesh,
        out_type=jax.ShapeDtypeStruct((_SCMID * 16,), jnp.float32),
        scratch_types=[
            pltpu.VMEM((_SCWIN, _D), jnp.float32),
            pltpu.VMEM((_SCWIN, _D), jnp.float32),
            pltpu.VMEM((_SCWIN * 16,), jnp.float32),
            pltpu.VMEM((_D,), jnp.float32),
            pltpu.SemaphoreType.DMA,
            pltpu.SemaphoreType.DMA,
        ],
    )
    return fn(table, wb)


def _fold_body(p_ref, m_ref, out_ref):
    out_ref[...] = jax.lax.dot_general(
        p_ref[...], m_ref[...], (((1,), (0,)), ((), ())),
        preferred_element_type=jnp.float32)


def _fold16(partials2d, mconst):
    return pl.pallas_call(
        _fold_body,
        grid=(_PROWS // 2048,),
        in_specs=[
            pl.BlockSpec((2048, 128), lambda i: (i, 0)),
            pl.BlockSpec((128, 128), lambda i: (0, 0)),
        ],
        out_specs=pl.BlockSpec((2048, 128), lambda i: (i, 0)),
        out_shape=jax.ShapeDtypeStruct((_PROWS, 128), jnp.float32),
    )(partials2d, mconst)


_MFOLD = np.asarray(
    (np.arange(128)[:, None] // 16) == np.arange(128)[None, :],
    dtype=np.float32)


def _sc_body(xr1_hbm, xr2_hbm, ht_hbm, mid_hbm, bv_hbm, out_hbm,
             idx_v, g_v, res_v, b_v, sem):
    wid = lax.axis_index("s") * 2 + lax.axis_index("c")
    base = wid * _RPW
    pltpu.sync_copy(bv_hbm, b_v)
    zero = jnp.zeros((16,), jnp.float32)

    def body(s, accs):
        off = s * _RPW
        return tuple(
            accs[rb] + g_v[pl.ds(off + rb * 16, 16)] for rb in range(8)
        )

    pltpu.sync_copy(xr1_hbm.at[wid], idx_v)
    pltpu.async_copy(ht_hbm.at[idx_v], g_v, sem).wait()
    acc1 = lax.fori_loop(0, _SEQ, body, (zero,) * 8)
    pltpu.sync_copy(xr2_hbm.at[wid], idx_v)
    pltpu.async_copy(mid_hbm.at[idx_v], g_v, sem).wait()
    acc2 = lax.fori_loop(0, _SEQ, body, (zero,) * 8)

    inv = jnp.float32(1.0 / _SEQ)
    bb = b_v[...]
    for rb in range(8):
        z = (acc1[rb] + acc2[rb]) * inv + bb
        res_v[pl.ds(rb * 16, 16)] = 1.0 / (1.0 + jnp.exp(-z))
    pltpu.sync_copy(res_v, out_hbm.at[pl.ds(base, _RPW)])


def _gather_pool(xr1, xr2, ht, mid_flat, bvec):
    mesh = plsc.VectorSubcoreMesh(core_axis_name="c", subcore_axis_name="s")
    fn = pl.kernel(
        _sc_body,
        mesh=mesh,
        out_type=jax.ShapeDtypeStruct((_BATCH,), jnp.float32),
        scratch_types=[
            pltpu.VMEM((_IPW,), jnp.int32),
            pltpu.VMEM((_IPW,), jnp.float32),
            pltpu.VMEM((_RPW,), jnp.float32),
            pltpu.VMEM((16,), jnp.float32),
            pltpu.SemaphoreType.DMA,
        ],
    )
    return fn(xr1, xr2, ht, mid_flat, bvec)


def _rearrange(ix):
    return ix.reshape(_NW, _RPW, _SEQ).transpose(0, 2, 1).reshape(_NW, _IPW)


def kernel(x, table, W, b):
    wb = W.reshape(_D)
    partials = _project_sc(table, wb)
    t_head = _project_tc(table, W, 31, 0)
    t_tail = _project_tc(table, W, 1, 61)
    ht = jnp.concatenate(
        [t_head.reshape(-1), t_tail.reshape(-1),
         jnp.zeros((_ZPAD,), jnp.float32)])
    mid = _fold16(partials.reshape(_PROWS, 128), jnp.asarray(_MFOLD))

    in_mid = (x >= _SCSTART) & (x < _SCEND)
    idx_ht = jnp.where(in_mid, _HTLEN + (x & (_ZPAD - 1)),
                       jnp.where(x < _SCSTART, x, x - _SCEND + _SCSTART))
    vp = x - _SCSTART
    idx_mid = jnp.where(in_mid, (vp >> 3) * 128 + (vp & 7),
                        (x & 32767) * 128 + 8 + (x & 63))

    bvec = jnp.full((16,), b[0], jnp.float32)
    y = _gather_pool(_rearrange(idx_ht), _rearrange(idx_mid),
                     ht, mid.reshape(-1), bvec)
    return y.reshape(_BATCH, 1)


# speedup vs baseline: 8.0668x; 1.2025x over previous
import functools

import jax
import jax.numpy as jnp
from jax import lax
from jax.experimental import pallas as pl
from jax.experimental.pallas import tpu as pltpu
from jax.experimental.pallas import tpu_sc as plsc

_VOCAB = 1_000_000
_D = 64
_BATCH = 4096
_SEQ = 200

_TROWS = 7936
_TBLK = 128
_GRID = _TROWS // _TBLK
_NW = 32
_RPW = _BATCH // _NW
_IPW = _SEQ * _RPW


def _proj_body(tbl_a, tbl_b, w_ref, b_ref, out_ref):
    w = w_ref[0, :]
    for half, ref in ((0, tbl_a), (1, tbl_b)):
        r3 = ref[...].reshape(_TBLK, 128, _D)
        out_ref[pl.ds(half * _TBLK, _TBLK), :] = (
            jnp.sum(r3 * w[None, None, :], axis=2) + b_ref[0])


def _project(table, w, b):
    return pl.pallas_call(
        _proj_body,
        grid=(_GRID // 2,),
        in_specs=[
            pl.BlockSpec((_TBLK * 128, _D), lambda i: (2 * i, 0)),
            pl.BlockSpec((_TBLK * 128, _D), lambda i: (2 * i + 1, 0)),
            pl.BlockSpec((1, _D), lambda i: (0, 0)),
            pl.BlockSpec(memory_space=pltpu.SMEM),
        ],
        out_specs=pl.BlockSpec((2 * _TBLK, 128), lambda i: (i, 0)),
        out_shape=jax.ShapeDtypeStruct((_TROWS, 128), jnp.float32),
    )(table, table, w, b)


def _sc_body(xr_hbm, t_hbm, out_hbm, idx_v, g_v, res_v, sem):
    wid = lax.axis_index("s") * 2 + lax.axis_index("c")
    base = wid * _RPW
    pltpu.sync_copy(xr_hbm.at[wid], idx_v)
    pltpu.async_copy(t_hbm.at[idx_v], g_v, sem).wait()

    zero = jnp.zeros((16,), jnp.float32)

    def body(s, accs):
        off = s * _RPW
        return tuple(
            accs[rb] + g_v[pl.ds(off + rb * 16, 16)] for rb in range(8)
        )

    accs = lax.fori_loop(0, _SEQ, body, (zero,) * 8)
    inv = jnp.float32(1.0 / _SEQ)
    for rb in range(8):
        z = accs[rb] * inv
        res_v[pl.ds(rb * 16, 16)] = 1.0 / (1.0 + jnp.exp(-z))
    pltpu.sync_copy(res_v, out_hbm.at[pl.ds(base, _RPW)])


def _gather_pool(xr, t_flat):
    mesh = plsc.VectorSubcoreMesh(core_axis_name="c", subcore_axis_name="s")
    fn = pl.kernel(
        _sc_body,
        mesh=mesh,
        out_type=jax.ShapeDtypeStruct((_BATCH,), jnp.float32),
        scratch_types=[
            pltpu.VMEM((_IPW,), jnp.int32),
            pltpu.VMEM((_IPW,), jnp.float32),
            pltpu.VMEM((_RPW,), jnp.float32),
            pltpu.SemaphoreType.DMA,
        ],
    )
    return fn(xr, t_flat)


def kernel(x, table, W, b):
    t2d = _project(table, W, b)
    t_flat = t2d.reshape(-1)
    xr = x.reshape(_NW, _RPW, _SEQ).transpose(0, 2, 1).reshape(_NW, _IPW)
    y = _gather_pool(xr, t_flat)
    return y.reshape(_BATCH, 1)
